# Initial kernel scaffold; baseline (speedup 1.0000x reference)
import jax
import jax.numpy as jnp
from jax.experimental import pallas as pl


def kernel(pt, angles, msg_W1, msg_b1, msg_bn_g, msg_bn_b, msg_W2, msg_b2, ro_W1, ro_b1, ro_bn1_g, ro_bn1_b, ro_W2, ro_b2, ro_bn2_g, ro_bn2_b, ro_W3, ro_b3):
    raise NotImplementedError("write your pallas kernel here")


# speedup vs baseline: 139.6687x; 139.6687x over previous
import jax
import jax.numpy as jnp
import numpy as np
from jax.experimental import pallas as pl
from jax.experimental.pallas import tpu as pltpu

_B = 500
_NP = 100
_G = 25
_STEPS = _B // _G
_DRN2 = np.float32(0.4 * 0.4)
_NEG = np.float32(0.01)
_TWO_PI = np.float32(2.0 * np.pi)
_EPS = np.float32(1e-5)


def _pairwise(ax, ay):
    r = jnp.sqrt(ax * ax + ay * ay)
    axi = ax[:, :, None]
    axj = ax[:, None, :]
    ayi = ay[:, :, None]
    ayj = ay[:, None, :]
    dx = axi - axj
    dy = ayi - ayj
    m = (dx * dx + dy * dy) <= _DRN2
    ri = r[:, :, None]
    rj = r[:, None, :]
    inv = 1.0 / (ri * rj)
    cosv = (axi * axj + ayi * ayj) * inv
    sinv = (ayi * axj - axi * ayj) * inv
    return ri, rj, cosv, sinv, m


def _stats_kernel(ax_ref, ay_ref, p1_ref, out_ref):
    ri, rj, cosv, sinv, m = _pairwise(ax_ref[0], ay_ref[0])
    cnt = jnp.sum(m.astype(jnp.float32))
    lane = jax.lax.broadcasted_iota(jnp.int32, (1, 128), 1)
    acc = cnt * (lane == 20).astype(jnp.float32)
    for c in range(10):
        h = (ri * p1_ref[0, c] + rj * p1_ref[1, c]
             + cosv * p1_ref[2, c] + sinv * p1_ref[3, c] + p1_ref[4, c])
        hm = jnp.where(m, h, 0.0)
        acc = acc + (jnp.sum(hm) * (lane == c).astype(jnp.float32)
                     + jnp.sum(hm * h) * (lane == 10 + c).astype(jnp.float32))

    @pl.when(pl.program_id(0) == 0)
    def _():
        out_ref[...] = acc

    @pl.when(pl.program_id(0) != 0)
    def _():
        out_ref[...] = out_ref[...] + acc


def _main_kernel(pt_ref, ax_ref, ay_ref, p2_ref, w2_ref, out_ref):
    ax = ax_ref[0]
    ay = ay_ref[0]
    ri, rj, cosv, sinv, m = _pairwise(ax, ay)
    deg = jnp.sum(m.astype(jnp.float32), axis=2)
    su = []
    for c in range(10):
        u = (ri * p2_ref[0, c] + rj * p2_ref[1, c]
             + cosv * p2_ref[2, c] + sinv * p2_ref[3, c] + p2_ref[4, c])
        u = jnp.where(u >= 0, u, _NEG * u)
        u = u.astype(jnp.bfloat16).astype(jnp.float32)
        su.append(jnp.sum(jnp.where(m, u, 0.0), axis=2))
    invdeg = 1.0 / deg
    means = []
    for c2 in range(11):
        t = deg * w2_ref[10, c2]
        for c in range(10):
            t = t + su[c] * w2_ref[c, c2]
        means.append(t * invdeg)
    theta = _TWO_PI * means[10]
    ct = jnp.cos(theta)
    st = jnp.sin(theta)
    rot0 = ct * ax - st * ay
    rot1 = st * ax + ct * ay
    ptb = pt_ref[0]
    w = ptb / jnp.sum(ptb, axis=1, keepdims=True)
    feats = means[:10] + [rot0, rot1]
    cols = [jnp.sum(w * f, axis=1)[:, None] for f in feats]
    out_ref[0] = jnp.concatenate(cols, axis=1)


def _readout_kernel(xg_ref, w1_ref, b1_ref, g1_ref, bb1_ref,
                    w2_ref, b2_ref, g2_ref, bb2_ref, w3_ref, b3_ref, out_ref):
    bf16 = jnp.bfloat16
    f32 = jnp.float32
    xg = xg_ref[...]
    h = jnp.dot(xg.astype(bf16), w1_ref[...].astype(bf16),
                preferred_element_type=f32) + b1_ref[...]
    mu = jnp.mean(h, axis=0, keepdims=True)
    var = jnp.mean((h - mu) ** 2, axis=0, keepdims=True)
    h = g1_ref[...] * (h - mu) / jnp.sqrt(var + _EPS) + bb1_ref[...]
    h = jnp.where(h >= 0, h, _NEG * h)
    h = jnp.dot(h.astype(bf16), w2_ref[...].astype(bf16),
                preferred_element_type=f32) + b2_ref[...]
    mu = jnp.mean(h, axis=0, keepdims=True)
    var = jnp.mean((h - mu) ** 2, axis=0, keepdims=True)
    h = g2_ref[...] * (h - mu) / jnp.sqrt(var + _EPS) + bb2_ref[...]
    h = jnp.where(h >= 0, h, _NEG * h)
    out = jnp.dot(h.astype(bf16), w3_ref[...].astype(bf16),
                  preferred_element_type=f32) + b3_ref[...]
    gamma = jnp.arctan2(xg[:, 11:12], xg[:, 10:11])
    out_ref[...] = jnp.concatenate([out, gamma], axis=1)


def kernel(pt, angles, msg_W1, msg_b1, msg_bn_g, msg_bn_b, msg_W2, msg_b2,
           ro_W1, ro_b1, ro_bn1_g, ro_bn1_b, ro_W2, ro_b2, ro_bn2_g, ro_bn2_b,
           ro_W3, ro_b3):
    f32 = jnp.float32
    axr = angles[:, :, 0].reshape(_STEPS, _G, _NP)
    ayr = angles[:, :, 1].reshape(_STEPS, _G, _NP)
    ptr = pt.reshape(_STEPS, _G, _NP)

    a1 = msg_W1[0] - msg_W1[2]
    b1c = msg_W1[1] + msg_W1[2]
    p1 = jnp.stack([a1, b1c, msg_W1[3], msg_W1[4], msg_b1])

    blk3 = pl.BlockSpec((1, _G, _NP), lambda i: (i, 0, 0))
    smem = pl.BlockSpec(memory_space=pltpu.SMEM)

    stats = pl.pallas_call(
        _stats_kernel,
        grid=(_STEPS,),
        in_specs=[blk3, blk3, smem],
        out_specs=pl.BlockSpec((1, 128), lambda i: (0, 0)),
        out_shape=jax.ShapeDtypeStruct((1, 128), f32),
    )(axr, ayr, p1)

    s = stats[0]
    cnt = s[20]
    mu = s[0:10] / cnt
    var = s[10:20] / cnt - mu * mu
    scale = msg_bn_g / jnp.sqrt(var + _EPS)
    shift = msg_bn_b - mu * scale
    p2 = jnp.stack([a1 * scale, b1c * scale, msg_W1[3] * scale,
                    msg_W1[4] * scale, msg_b1 * scale + shift])
    w2r = msg_W2.astype(jnp.bfloat16).astype(f32)
    w2e = jnp.concatenate([w2r, msg_b2[None, :]], axis=0)

    xg = pl.pallas_call(
        _main_kernel,
        grid=(_STEPS,),
        in_specs=[blk3, blk3, blk3, smem, smem],
        out_specs=pl.BlockSpec((1, _G, 12), lambda i: (i, 0, 0)),
        out_shape=jax.ShapeDtypeStruct((_STEPS, _G, 12), f32),
    )(ptr, axr, ayr, p2, w2e)

    out = pl.pallas_call(
        _readout_kernel,
        out_shape=jax.ShapeDtypeStruct((_B, 2), f32),
    )(xg.reshape(_B, 12), ro_W1, ro_b1[None, :], ro_bn1_g[None, :],
      ro_bn1_b[None, :], ro_W2, ro_b2[None, :], ro_bn2_g[None, :],
      ro_bn2_b[None, :], ro_W3, ro_b3[None, :])
    return out
